# baseline (speedup 1.0000x reference)
import functools

import jax
import jax.numpy as jnp
from jax import lax
from jax.experimental import pallas as pl
from jax.experimental.pallas import tpu as pltpu
from jax.experimental.pallas import tpu_sc as plsc

LANES = 16
ROWS_SC = 2048


def _sc_body(n_cat, rows_w, n_cores,
             x_hbm, tab_hbm, min_hbm, out_hbm,
             x_v, out_v, tab_v, min_v):
    wid = lax.axis_index("s") * n_cores + lax.axis_index("c")
    base = wid * rows_w
    pltpu.sync_copy(x_hbm.at[pl.ds(base, rows_w)], x_v)
    pltpu.sync_copy(tab_hbm, tab_v)
    pltpu.sync_copy(min_hbm, min_v)

    c0 = lax.iota(jnp.int32, LANES)
    c1 = c0 + (n_cat - LANES)
    m0 = plsc.load_gather(min_v, [c0])
    m1 = plsc.load_gather(min_v, [c1])
    s1 = n_cat - LANES

    @plsc.parallel_loop(0, rows_w)
    def _(r):
        v0 = x_v[r, pl.ds(0, LANES)] - m0
        out_v[r, pl.ds(0, LANES)] = plsc.load_gather(tab_v, [c0, v0])
        v1 = x_v[r, pl.ds(s1, LANES)] - m1
        out_v[r, pl.ds(s1, LANES)] = plsc.load_gather(tab_v, [c1, v1])

    pltpu.sync_copy(out_v, out_hbm.at[pl.ds(base, rows_w)])


def _sc_run(x_sc, map_table, min_vals):
    rows, n_cat = x_sc.shape
    info = plsc.get_sparse_core_info()
    n_workers = info.num_cores * info.num_subcores
    rows_w = rows // n_workers
    assert rows == rows_w * n_workers and rows_w % 8 == 0

    mesh = plsc.VectorSubcoreMesh(core_axis_name="c", subcore_axis_name="s")
    body = functools.partial(_sc_body, n_cat, rows_w, info.num_cores)
    run = pl.kernel(
        body,
        out_type=jax.ShapeDtypeStruct((rows, n_cat), jnp.int32),
        mesh=mesh,
        scratch_types=[
            pltpu.VMEM((rows_w, n_cat), jnp.int32),
            pltpu.VMEM((rows_w, n_cat), jnp.int32),
            pltpu.VMEM(map_table.shape, jnp.int32),
            pltpu.VMEM(min_vals.shape, jnp.int32),
        ],
        compiler_params=pltpu.CompilerParams(needs_layout_passes=False,
                                             skip_device_barrier=True),
    )
    return run(x_sc, map_table, min_vals)


def _tc_body(xT_ref, tab_ref, min_ref, outT_ref):
    v = xT_ref[...] - min_ref[...]
    outT_ref[...] = jnp.take_along_axis(
        tab_ref[...], v, axis=1, mode="promise_in_bounds")


def _tc_run(x_tc, map_table, min_vals):
    rows, n_cat = x_tc.shape
    run = pl.pallas_call(
        _tc_body,
        out_shape=jax.ShapeDtypeStruct((n_cat, rows), jnp.int32),
    )
    return run(x_tc.T, map_table, min_vals[:, None]).T


def kernel(x, map_table, min_vals):
    batch, n_cat = x.shape
    assert LANES < n_cat <= 2 * LANES and batch > ROWS_SC
    out_sc = _sc_run(x[:ROWS_SC], map_table, min_vals)
    out_tc = _tc_run(x[ROWS_SC:], map_table, min_vals)
    return jnp.concatenate([out_sc, out_tc], axis=0)


# speedup vs baseline: 1.0066x; 1.0066x over previous
import functools

import jax
import jax.numpy as jnp
from jax import lax
from jax.experimental import pallas as pl
from jax.experimental.pallas import tpu as pltpu
from jax.experimental.pallas import tpu_sc as plsc

LANES = 16
ROWS_SC = 4096


def _sc_body(n_cat, rows_w, n_cores,
             x_hbm, tab_hbm, min_hbm, out_hbm,
             x_v, out_v, tab_v, min_v):
    wid = lax.axis_index("s") * n_cores + lax.axis_index("c")
    base = wid * rows_w
    pltpu.sync_copy(x_hbm.at[pl.ds(base, rows_w)], x_v)
    pltpu.sync_copy(tab_hbm, tab_v)
    pltpu.sync_copy(min_hbm, min_v)

    c0 = lax.iota(jnp.int32, LANES)
    c1 = c0 + (n_cat - LANES)
    m0 = plsc.load_gather(min_v, [c0])
    m1 = plsc.load_gather(min_v, [c1])
    s1 = n_cat - LANES

    @plsc.parallel_loop(0, rows_w)
    def _(r):
        v0 = x_v[r, pl.ds(0, LANES)] - m0
        out_v[r, pl.ds(0, LANES)] = plsc.load_gather(tab_v, [c0, v0])
        v1 = x_v[r, pl.ds(s1, LANES)] - m1
        out_v[r, pl.ds(s1, LANES)] = plsc.load_gather(tab_v, [c1, v1])

    pltpu.sync_copy(out_v, out_hbm.at[pl.ds(base, rows_w)])


def _sc_run(x_sc, map_table, min_vals):
    rows, n_cat = x_sc.shape
    info = plsc.get_sparse_core_info()
    n_workers = info.num_cores * info.num_subcores
    rows_w = rows // n_workers
    assert rows == rows_w * n_workers and rows_w % 8 == 0

    mesh = plsc.VectorSubcoreMesh(core_axis_name="c", subcore_axis_name="s")
    body = functools.partial(_sc_body, n_cat, rows_w, info.num_cores)
    run = pl.kernel(
        body,
        out_type=jax.ShapeDtypeStruct((rows, n_cat), jnp.int32),
        mesh=mesh,
        scratch_types=[
            pltpu.VMEM((rows_w, n_cat), jnp.int32),
            pltpu.VMEM((rows_w, n_cat), jnp.int32),
            pltpu.VMEM(map_table.shape, jnp.int32),
            pltpu.VMEM(min_vals.shape, jnp.int32),
        ],
        compiler_params=pltpu.CompilerParams(needs_layout_passes=False,
                                             skip_device_barrier=True),
    )
    return run(x_sc, map_table, min_vals)


def _tc_body(xT_ref, tab_ref, min_ref, outT_ref):
    v = xT_ref[...] - min_ref[...]
    outT_ref[...] = jnp.take_along_axis(
        tab_ref[...], v, axis=1, mode="promise_in_bounds")


def _tc_run(x_tc, map_table, min_vals):
    rows, n_cat = x_tc.shape
    run = pl.pallas_call(
        _tc_body,
        out_shape=jax.ShapeDtypeStruct((n_cat, rows), jnp.int32),
    )
    return run(x_tc.T, map_table, min_vals[:, None]).T


def kernel(x, map_table, min_vals):
    batch, n_cat = x.shape
    assert LANES < n_cat <= 2 * LANES and batch > ROWS_SC
    out_sc = _sc_run(x[:ROWS_SC], map_table, min_vals)
    out_tc = _tc_run(x[ROWS_SC:], map_table, min_vals)
    return jnp.concatenate([out_sc, out_tc], axis=0)
